# baseline (speedup 1.0000x reference)
import functools
import jax
import jax.numpy as jnp
from jax import lax
from jax.experimental import pallas as pl
from jax.experimental.pallas import tpu as pltpu, tpu_sc as plsc

_B, _N, _E = 16, 1000, 128
_NP = 1024
_RC = 200
_NC = _N // _RC
_K = 10
_NW = 32
_RPW = (_B * _N) // _NW


def _dist_body(xq_ref, xbT_ref, dist_ref):
    q0 = xq_ref[0, :, 0:1]
    q1 = xq_ref[0, :, 1:2]
    k0 = xbT_ref[0, 0:1, :]
    k1 = xbT_ref[0, 1:2, :]
    d0 = q0 - k0
    d1 = q1 - k1
    dist2 = d0 * d0 + d1 * d1
    pad = jnp.full((_RC, _NP - _N), jnp.inf, jnp.float32)
    dist_ref[0] = jnp.concatenate([dist2, pad], axis=1)


def _sc_select_body(dist_hbm, x0x_hbm, x0y_hbm, gx_hbm, gy_hbm,
                    row_v, x0x_v, x0y_v, gx_v, gy_v):
    wid = lax.axis_index("s") * 2 + lax.axis_index("c")
    pltpu.sync_copy(x0x_hbm, x0x_v)
    pltpu.sync_copy(x0y_hbm, x0y_v)
    lane = lax.iota(jnp.int32, 16)
    inf16 = jnp.full((16,), jnp.inf, jnp.float32)
    zero16 = jnp.zeros((16,), jnp.float32)
    base = wid * _RPW

    def row_fn(r, carry):
        pltpu.sync_copy(dist_hbm.at[base + r], row_v)

        def chunk_fn(j, kc):
            ck, ci = kc
            cv = row_v[pl.ds(j * 16, 16)]
            cidx = lane + j * 16
            sk, si = plsc.sort_key_val(cv, cidx)
            rk = lax.rev(sk, (0,))
            ri = lax.rev(si, (0,))
            sel = ck <= rk
            mk = jnp.where(sel, ck, rk)
            mi = jnp.where(sel, ci, ri)
            mk2, mi2 = plsc.sort_key_val(mk, mi)
            mk2 = jnp.where(lane < _K, mk2, inf16)
            return mk2, mi2

        ck, ci = lax.fori_loop(0, _NP // 16, chunk_fn,
                               (inf16, jnp.zeros((16,), jnp.int32)))
        safe = jnp.where(lane < _K, ci, 0)
        gx = plsc.load_gather(x0x_v, [safe])
        gy = plsc.load_gather(x0y_v, [safe])
        gx_v[pl.ds(r * 16, 16)] = jnp.where(lane < _K, gx, zero16)
        gy_v[pl.ds(r * 16, 16)] = jnp.where(lane < _K, gy, zero16)
        return carry

    lax.fori_loop(0, _RPW, row_fn, 0)
    pltpu.sync_copy(gx_v, gx_hbm.at[wid])
    pltpu.sync_copy(gy_v, gy_hbm.at[wid])


def _fe_body(xq_ref, gx_ref, gy_ref, wc_ref, fe_ref, stats_ref):
    b = pl.program_id(0)
    c = pl.program_id(1)
    q0 = xq_ref[0, :, 0:1]
    q1 = xq_ref[0, :, 1:2]
    s0 = jnp.sum(gx_ref[0], axis=1, keepdims=True)
    s1 = jnp.sum(gy_ref[0], axis=1, keepdims=True)
    t0 = s0 - jnp.float32(_K) * q0
    t1 = s1 - jnp.float32(_K) * q1
    fe = (q0 * wc_ref[0:1, :] + q1 * wc_ref[1:2, :]
          + t0 * wc_ref[2:3, :] + t1 * wc_ref[3:4, :])

    fe_ref[0, :, :] = fe

    @pl.when((b == 0) & (c == 0))
    def _():
        stats_ref[:, :] = jnp.zeros((8, _E), jnp.float32)

    stats_ref[0:1, :] += jnp.sum(fe, axis=0, keepdims=True)
    stats_ref[1:2, :] += jnp.sum(fe * fe, axis=0, keepdims=True)


def _bn_body(fe_ref, stats_ref, dep_ref, wdep_ref, bdep_ref, bnw_ref, bnb_ref,
             hb_ref, hd_ref, mh_ref):
    inv_n = jnp.float32(1.0 / (_B * _N))
    mean = stats_ref[0:1, :] * inv_n
    ex2 = stats_ref[1:2, :] * inv_n
    var = ex2 - mean * mean
    scale = jax.lax.rsqrt(var + jnp.float32(1e-5)) * bnw_ref[0:1, :]
    fe = fe_ref[0]
    normed = (fe - mean) * scale + bnb_ref[0:1, :]
    hb = jnp.where(normed >= 0, normed, jnp.float32(0.01) * normed)
    hb_ref[0] = hb
    dd0 = dep_ref[0, :, 0:1]
    dd1 = dep_ref[0, :, 1:2]
    dep = dd0 * wdep_ref[0:1, :] + dd1 * wdep_ref[1:2, :] + bdep_ref[0:1, :]
    hd = jnp.where(dep >= 0, dep, jnp.float32(0.01) * dep)
    hd_ref[0] = hd
    mh_ref[0] = (jnp.sum(hb, axis=0, keepdims=True) + hd) / jnp.float32(_N + 1)


def kernel(loc, depot, W_init, b_init, W_nbr, b_nbr, W_fin, b_fin,
           W_dep, b_dep, bn_w, bn_b):
    locT = jnp.transpose(loc, (0, 2, 1))
    wc = jnp.concatenate([W_init @ W_fin, W_nbr @ W_fin], axis=0)

    dist = pl.pallas_call(
        _dist_body,
        grid=(_B, _NC),
        in_specs=[
            pl.BlockSpec((1, _RC, 2), lambda b, c: (b, c, 0)),
            pl.BlockSpec((1, 2, _N), lambda b, c: (b, 0, 0)),
        ],
        out_specs=pl.BlockSpec((1, _RC, _NP), lambda b, c: (b, c, 0)),
        out_shape=jax.ShapeDtypeStruct((_B, _N, _NP), jnp.float32),
    )(loc, locT)

    mesh = plsc.VectorSubcoreMesh(core_axis_name="c", subcore_axis_name="s")
    x0x = jnp.pad(loc[0, :, 0], (0, _NP - _N))
    x0y = jnp.pad(loc[0, :, 1], (0, _NP - _N))
    sc_fn = functools.partial(
        pl.kernel, mesh=mesh,
        compiler_params=pltpu.CompilerParams(needs_layout_passes=False),
        out_type=[
            jax.ShapeDtypeStruct((_NW, _RPW * 16), jnp.float32),
            jax.ShapeDtypeStruct((_NW, _RPW * 16), jnp.float32),
        ],
        scratch_types=[
            pltpu.VMEM((_NP,), jnp.float32),
            pltpu.VMEM((_NP,), jnp.float32),
            pltpu.VMEM((_NP,), jnp.float32),
            pltpu.VMEM((_RPW * 16,), jnp.float32),
            pltpu.VMEM((_RPW * 16,), jnp.float32),
        ],
    )(_sc_select_body)
    gx_raw, gy_raw = sc_fn(dist.reshape(_B * _N, _NP), x0x, x0y)
    gx = gx_raw.reshape(_B, _N, 16)
    gy = gy_raw.reshape(_B, _N, 16)

    fe, stats = pl.pallas_call(
        _fe_body,
        grid=(_B, _NC),
        in_specs=[
            pl.BlockSpec((1, _RC, 2), lambda b, c: (b, c, 0)),
            pl.BlockSpec((1, _RC, 16), lambda b, c: (b, c, 0)),
            pl.BlockSpec((1, _RC, 16), lambda b, c: (b, c, 0)),
            pl.BlockSpec((4, _E), lambda b, c: (0, 0)),
        ],
        out_specs=[
            pl.BlockSpec((1, _RC, _E), lambda b, c: (b, c, 0)),
            pl.BlockSpec((8, _E), lambda b, c: (0, 0)),
        ],
        out_shape=[
            jax.ShapeDtypeStruct((_B, _N, _E), jnp.float32),
            jax.ShapeDtypeStruct((8, _E), jnp.float32),
        ],
    )(loc, gx, gy, wc)

    hb, hd, mh = pl.pallas_call(
        _bn_body,
        grid=(_B,),
        in_specs=[
            pl.BlockSpec((1, _N, _E), lambda b: (b, 0, 0)),
            pl.BlockSpec((8, _E), lambda b: (0, 0)),
            pl.BlockSpec((1, 1, 2), lambda b: (b, 0, 0)),
            pl.BlockSpec((2, _E), lambda b: (0, 0)),
            pl.BlockSpec((1, _E), lambda b: (0, 0)),
            pl.BlockSpec((1, _E), lambda b: (0, 0)),
            pl.BlockSpec((1, _E), lambda b: (0, 0)),
        ],
        out_specs=[
            pl.BlockSpec((1, _N, _E), lambda b: (b, 0, 0)),
            pl.BlockSpec((1, 1, _E), lambda b: (b, 0, 0)),
            pl.BlockSpec((1, 1, _E), lambda b: (b, 0, 0)),
        ],
        out_shape=[
            jax.ShapeDtypeStruct((_B, _N, _E), jnp.float32),
            jax.ShapeDtypeStruct((_B, 1, _E), jnp.float32),
            jax.ShapeDtypeStruct((_B, 1, _E), jnp.float32),
        ],
    )(fe, stats, depot, W_dep, b_dep[None, :], bn_w[None, :], bn_b[None, :])

    h = jnp.concatenate([hd, hb], axis=1)
    return h, mh[:, 0, :]


# speedup vs baseline: 3.1472x; 3.1472x over previous
import functools
import jax
import jax.numpy as jnp
from jax import lax
from jax.experimental import pallas as pl
from jax.experimental.pallas import tpu as pltpu, tpu_sc as plsc

_B, _N, _E = 16, 1000, 128
_RC = 200
_NC = _N // _RC
_K = 10
_NP = 1024
_BSC = 4
_BTC = _B - _BSC
_NW = 32
_RPW = (_BSC * _N) // _NW


def _knn_fe_body(xq_ref, xbT_ref, x0_ref, wc_ref, fe_ref, stats_ref):
    b = pl.program_id(0)
    c = pl.program_id(1)
    q0 = xq_ref[0, :, 0:1]
    q1 = xq_ref[0, :, 1:2]
    k0 = xbT_ref[0, 0:1, :]
    k1 = xbT_ref[0, 1:2, :]
    d0 = q0 - k0
    d1 = q1 - k1
    dist2 = d0 * d0 + d1 * d1
    iota = jax.lax.broadcasted_iota(jnp.int32, (_RC, _N), 1).astype(jnp.float32)
    big = jnp.float32(jnp.inf)
    bigi = jnp.float32(2e9)
    zero = jnp.float32(0.0)
    one = jnp.float32(1.0)

    def extract(work, m):
        cand = jnp.where(work == m, iota, bigi)
        idx = jnp.min(cand, axis=1, keepdims=True)
        return jnp.where(cand == idx, big, work)

    work = extract(dist2, zero)
    for _ in range(_K - 1):
        m = jnp.min(work, axis=1, keepdims=True)
        work = extract(work, m)

    acc = jnp.where(work == big, one, zero)
    s = jnp.dot(acc, x0_ref[0], preferred_element_type=jnp.float32)
    t0 = s[:, 0:1] - jnp.float32(_K) * q0
    t1 = s[:, 1:2] - jnp.float32(_K) * q1
    fe = (q0 * wc_ref[0:1, :] + q1 * wc_ref[1:2, :]
          + t0 * wc_ref[2:3, :] + t1 * wc_ref[3:4, :])
    fe_ref[0, :, :] = fe

    @pl.when((b == 0) & (c == 0))
    def _():
        stats_ref[:, :] = jnp.zeros((8, _E), jnp.float32)

    stats_ref[0:1, :] += jnp.sum(fe, axis=0, keepdims=True)
    stats_ref[1:2, :] += jnp.sum(fe * fe, axis=0, keepdims=True)


def _dist_body(xq_ref, xbT_ref, dist_ref):
    q0 = xq_ref[0, :, 0:1]
    q1 = xq_ref[0, :, 1:2]
    k0 = xbT_ref[0, 0:1, :]
    k1 = xbT_ref[0, 1:2, :]
    d0 = q0 - k0
    d1 = q1 - k1
    dist2 = d0 * d0 + d1 * d1
    pad = jnp.full((_RC, _NP - _N), jnp.inf, jnp.float32)
    dist_ref[0] = jnp.concatenate([dist2, pad], axis=1)


def _sc_select_body(dist_hbm, x0x_hbm, x0y_hbm, gx_hbm, gy_hbm,
                    row_v, x0x_v, x0y_v, gx_v, gy_v):
    wid = lax.axis_index("s") * 2 + lax.axis_index("c")
    pltpu.sync_copy(x0x_hbm, x0x_v)
    pltpu.sync_copy(x0y_hbm, x0y_v)
    lane = lax.iota(jnp.int32, 16)
    inf16 = jnp.full((16,), jnp.inf, jnp.float32)
    zero16 = jnp.zeros((16,), jnp.float32)
    base = wid * _RPW

    def row_fn(r, carry):
        pltpu.sync_copy(dist_hbm.at[base + r], row_v)

        def chunk_fn(j, kc):
            ck, ci = kc
            cv = row_v[pl.ds(j * 16, 16)]
            cidx = lane + j * 16
            sk, si = plsc.sort_key_val(cv, cidx)
            rk = lax.rev(sk, (0,))
            ri = lax.rev(si, (0,))
            sel = ck <= rk
            mk = jnp.where(sel, ck, rk)
            mi = jnp.where(sel, ci, ri)
            mk2, mi2 = plsc.sort_key_val(mk, mi)
            mk2 = jnp.where(lane < _K, mk2, inf16)
            return mk2, mi2

        ck, ci = lax.fori_loop(0, _NP // 16, chunk_fn,
                               (inf16, jnp.zeros((16,), jnp.int32)))
        safe = jnp.where(lane < _K, ci, 0)
        gx = plsc.load_gather(x0x_v, [safe])
        gy = plsc.load_gather(x0y_v, [safe])
        gx_v[pl.ds(r * 16, 16)] = jnp.where(lane < _K, gx, zero16)
        gy_v[pl.ds(r * 16, 16)] = jnp.where(lane < _K, gy, zero16)
        return carry

    lax.fori_loop(0, _RPW, row_fn, 0)
    pltpu.sync_copy(gx_v, gx_hbm.at[wid])
    pltpu.sync_copy(gy_v, gy_hbm.at[wid])


def _fe_body(xq_ref, gx_ref, gy_ref, wc_ref, fe_ref, stats_ref):
    b = pl.program_id(0)
    c = pl.program_id(1)
    q0 = xq_ref[0, :, 0:1]
    q1 = xq_ref[0, :, 1:2]
    s0 = jnp.sum(gx_ref[0], axis=1, keepdims=True)
    s1 = jnp.sum(gy_ref[0], axis=1, keepdims=True)
    t0 = s0 - jnp.float32(_K) * q0
    t1 = s1 - jnp.float32(_K) * q1
    fe = (q0 * wc_ref[0:1, :] + q1 * wc_ref[1:2, :]
          + t0 * wc_ref[2:3, :] + t1 * wc_ref[3:4, :])
    fe_ref[0, :, :] = fe

    @pl.when((b == 0) & (c == 0))
    def _():
        stats_ref[:, :] = jnp.zeros((8, _E), jnp.float32)

    stats_ref[0:1, :] += jnp.sum(fe, axis=0, keepdims=True)
    stats_ref[1:2, :] += jnp.sum(fe * fe, axis=0, keepdims=True)


def _bn_body(fe_ref, st1_ref, st2_ref, dep_ref, wdep_ref, bdep_ref,
             bnw_ref, bnb_ref, hb_ref, hd_ref, mh_ref):
    inv_n = jnp.float32(1.0 / (_B * _N))
    mean = (st1_ref[0:1, :] + st2_ref[0:1, :]) * inv_n
    ex2 = (st1_ref[1:2, :] + st2_ref[1:2, :]) * inv_n
    var = ex2 - mean * mean
    scale = jax.lax.rsqrt(var + jnp.float32(1e-5)) * bnw_ref[0:1, :]
    fe = fe_ref[0]
    normed = (fe - mean) * scale + bnb_ref[0:1, :]
    hb = jnp.where(normed >= 0, normed, jnp.float32(0.01) * normed)
    hb_ref[0] = hb
    dd0 = dep_ref[0, :, 0:1]
    dd1 = dep_ref[0, :, 1:2]
    dep = dd0 * wdep_ref[0:1, :] + dd1 * wdep_ref[1:2, :] + bdep_ref[0:1, :]
    hd = jnp.where(dep >= 0, dep, jnp.float32(0.01) * dep)
    hd_ref[0] = hd
    mh_ref[0] = (jnp.sum(hb, axis=0, keepdims=True) + hd) / jnp.float32(_N + 1)


def kernel(loc, depot, W_init, b_init, W_nbr, b_nbr, W_fin, b_fin,
           W_dep, b_dep, bn_w, bn_b):
    locT = jnp.transpose(loc, (0, 2, 1))
    wc = jnp.concatenate([W_init @ W_fin, W_nbr @ W_fin], axis=0)

    dist_sc = pl.pallas_call(
        _dist_body,
        grid=(_BSC, _NC),
        in_specs=[
            pl.BlockSpec((1, _RC, 2), lambda b, c: (b + _BTC, c, 0)),
            pl.BlockSpec((1, 2, _N), lambda b, c: (b + _BTC, 0, 0)),
        ],
        out_specs=pl.BlockSpec((1, _RC, _NP), lambda b, c: (b, c, 0)),
        out_shape=jax.ShapeDtypeStruct((_BSC, _N, _NP), jnp.float32),
    )(loc, locT)

    mesh = plsc.VectorSubcoreMesh(core_axis_name="c", subcore_axis_name="s")
    x0x = jnp.pad(loc[0, :, 0], (0, _NP - _N))
    x0y = jnp.pad(loc[0, :, 1], (0, _NP - _N))
    sc_fn = functools.partial(
        pl.kernel, mesh=mesh,
        compiler_params=pltpu.CompilerParams(needs_layout_passes=False),
        out_type=[
            jax.ShapeDtypeStruct((_NW, _RPW * 16), jnp.float32),
            jax.ShapeDtypeStruct((_NW, _RPW * 16), jnp.float32),
        ],
        scratch_types=[
            pltpu.VMEM((_NP,), jnp.float32),
            pltpu.VMEM((_NP,), jnp.float32),
            pltpu.VMEM((_NP,), jnp.float32),
            pltpu.VMEM((_RPW * 16,), jnp.float32),
            pltpu.VMEM((_RPW * 16,), jnp.float32),
        ],
    )(_sc_select_body)
    gx_raw, gy_raw = sc_fn(dist_sc.reshape(_BSC * _N, _NP), x0x, x0y)
    gx = gx_raw.reshape(_BSC, _N, 16)
    gy = gy_raw.reshape(_BSC, _N, 16)

    fe_tc, stats_tc = pl.pallas_call(
        _knn_fe_body,
        grid=(_BTC, _NC),
        in_specs=[
            pl.BlockSpec((1, _RC, 2), lambda b, c: (b, c, 0)),
            pl.BlockSpec((1, 2, _N), lambda b, c: (b, 0, 0)),
            pl.BlockSpec((1, _N, 2), lambda b, c: (0, 0, 0)),
            pl.BlockSpec((4, _E), lambda b, c: (0, 0)),
        ],
        out_specs=[
            pl.BlockSpec((1, _RC, _E), lambda b, c: (b, c, 0)),
            pl.BlockSpec((8, _E), lambda b, c: (0, 0)),
        ],
        out_shape=[
            jax.ShapeDtypeStruct((_BTC, _N, _E), jnp.float32),
            jax.ShapeDtypeStruct((8, _E), jnp.float32),
        ],
    )(loc, locT, loc, wc)

    fe_sc, stats_sc = pl.pallas_call(
        _fe_body,
        grid=(_BSC, _NC),
        in_specs=[
            pl.BlockSpec((1, _RC, 2), lambda b, c: (b + _BTC, c, 0)),
            pl.BlockSpec((1, _RC, 16), lambda b, c: (b, c, 0)),
            pl.BlockSpec((1, _RC, 16), lambda b, c: (b, c, 0)),
            pl.BlockSpec((4, _E), lambda b, c: (0, 0)),
        ],
        out_specs=[
            pl.BlockSpec((1, _RC, _E), lambda b, c: (b, c, 0)),
            pl.BlockSpec((8, _E), lambda b, c: (0, 0)),
        ],
        out_shape=[
            jax.ShapeDtypeStruct((_BSC, _N, _E), jnp.float32),
            jax.ShapeDtypeStruct((8, _E), jnp.float32),
        ],
    )(loc, gx, gy, wc)

    fe = jnp.concatenate([fe_tc, fe_sc], axis=0)

    hb, hd, mh = pl.pallas_call(
        _bn_body,
        grid=(_B,),
        in_specs=[
            pl.BlockSpec((1, _N, _E), lambda b: (b, 0, 0)),
            pl.BlockSpec((8, _E), lambda b: (0, 0)),
            pl.BlockSpec((8, _E), lambda b: (0, 0)),
            pl.BlockSpec((1, 1, 2), lambda b: (b, 0, 0)),
            pl.BlockSpec((2, _E), lambda b: (0, 0)),
            pl.BlockSpec((1, _E), lambda b: (0, 0)),
            pl.BlockSpec((1, _E), lambda b: (0, 0)),
            pl.BlockSpec((1, _E), lambda b: (0, 0)),
        ],
        out_specs=[
            pl.BlockSpec((1, _N, _E), lambda b: (b, 0, 0)),
            pl.BlockSpec((1, 1, _E), lambda b: (b, 0, 0)),
            pl.BlockSpec((1, 1, _E), lambda b: (b, 0, 0)),
        ],
        out_shape=[
            jax.ShapeDtypeStruct((_B, _N, _E), jnp.float32),
            jax.ShapeDtypeStruct((_B, 1, _E), jnp.float32),
            jax.ShapeDtypeStruct((_B, 1, _E), jnp.float32),
        ],
    )(fe, stats_tc, stats_sc, depot, W_dep, b_dep[None, :],
      bn_w[None, :], bn_b[None, :])

    h = jnp.concatenate([hd, hb], axis=1)
    return h, mh[:, 0, :]


# speedup vs baseline: 3.5450x; 1.1264x over previous
import jax
import jax.numpy as jnp
from jax.experimental import pallas as pl

_B, _N, _E = 16, 1000, 128
_RC = 200
_NC = _N // _RC
_K = 10


def _knn_fe_body(xq_ref, xbT_ref, x0_ref, wc_ref, fe_ref, stats_ref):
    b = pl.program_id(0)
    c = pl.program_id(1)
    q0 = xq_ref[0, :, 0:1]
    q1 = xq_ref[0, :, 1:2]
    k0 = xbT_ref[0, 0:1, :]
    k1 = xbT_ref[0, 1:2, :]
    d0 = q0 - k0
    d1 = q1 - k1
    dist2 = d0 * d0 + d1 * d1
    iota = jax.lax.broadcasted_iota(jnp.int32, (_RC, _N), 1).astype(jnp.float32)
    big = jnp.float32(jnp.inf)
    bigi = jnp.float32(2e9)
    zero = jnp.float32(0.0)
    one = jnp.float32(1.0)

    def extract(work, m):
        cand = jnp.where(work == m, iota, bigi)
        idx = jnp.min(cand, axis=1, keepdims=True)
        return jnp.where(cand == idx, big, work)

    work = extract(dist2, zero)
    for _ in range(_K - 1):
        m = jnp.min(work, axis=1, keepdims=True)
        work = extract(work, m)

    acc = jnp.where(work == big, one, zero)
    s = jnp.dot(acc, x0_ref[0], preferred_element_type=jnp.float32)
    t0 = s[:, 0:1] - jnp.float32(_K) * q0
    t1 = s[:, 1:2] - jnp.float32(_K) * q1
    fe = (q0 * wc_ref[0:1, :] + q1 * wc_ref[1:2, :]
          + t0 * wc_ref[2:3, :] + t1 * wc_ref[3:4, :])
    fe_ref[0, :, :] = fe

    @pl.when((b == 0) & (c == 0))
    def _():
        stats_ref[:, :] = jnp.zeros((8, _E), jnp.float32)

    stats_ref[0:1, :] += jnp.sum(fe, axis=0, keepdims=True)
    stats_ref[1:2, :] += jnp.sum(fe * fe, axis=0, keepdims=True)


def _bn_body(fe_ref, stats_ref, dep_ref, wdep_ref, bdep_ref, bnw_ref, bnb_ref,
             hb_ref, hd_ref, mh_ref):
    inv_n = jnp.float32(1.0 / (_B * _N))
    mean = stats_ref[0:1, :] * inv_n
    ex2 = stats_ref[1:2, :] * inv_n
    var = ex2 - mean * mean
    scale = jax.lax.rsqrt(var + jnp.float32(1e-5)) * bnw_ref[0:1, :]
    fe = fe_ref[0]
    normed = (fe - mean) * scale + bnb_ref[0:1, :]
    hb = jnp.where(normed >= 0, normed, jnp.float32(0.01) * normed)
    hb_ref[0] = hb
    dd0 = dep_ref[0, :, 0:1]
    dd1 = dep_ref[0, :, 1:2]
    dep = dd0 * wdep_ref[0:1, :] + dd1 * wdep_ref[1:2, :] + bdep_ref[0:1, :]
    hd = jnp.where(dep >= 0, dep, jnp.float32(0.01) * dep)
    hd_ref[0] = hd
    mh_ref[0] = (jnp.sum(hb, axis=0, keepdims=True) + hd) / jnp.float32(_N + 1)


def kernel(loc, depot, W_init, b_init, W_nbr, b_nbr, W_fin, b_fin,
           W_dep, b_dep, bn_w, bn_b):
    locT = jnp.transpose(loc, (0, 2, 1))
    wc = jnp.concatenate([W_init @ W_fin, W_nbr @ W_fin], axis=0)

    fe, stats = pl.pallas_call(
        _knn_fe_body,
        grid=(_B, _NC),
        in_specs=[
            pl.BlockSpec((1, _RC, 2), lambda b, c: (b, c, 0)),
            pl.BlockSpec((1, 2, _N), lambda b, c: (b, 0, 0)),
            pl.BlockSpec((1, _N, 2), lambda b, c: (0, 0, 0)),
            pl.BlockSpec((4, _E), lambda b, c: (0, 0)),
        ],
        out_specs=[
            pl.BlockSpec((1, _RC, _E), lambda b, c: (b, c, 0)),
            pl.BlockSpec((8, _E), lambda b, c: (0, 0)),
        ],
        out_shape=[
            jax.ShapeDtypeStruct((_B, _N, _E), jnp.float32),
            jax.ShapeDtypeStruct((8, _E), jnp.float32),
        ],
    )(loc, locT, loc, wc)

    hb, hd, mh = pl.pallas_call(
        _bn_body,
        grid=(_B,),
        in_specs=[
            pl.BlockSpec((1, _N, _E), lambda b: (b, 0, 0)),
            pl.BlockSpec((8, _E), lambda b: (0, 0)),
            pl.BlockSpec((1, 1, 2), lambda b: (b, 0, 0)),
            pl.BlockSpec((2, _E), lambda b: (0, 0)),
            pl.BlockSpec((1, _E), lambda b: (0, 0)),
            pl.BlockSpec((1, _E), lambda b: (0, 0)),
            pl.BlockSpec((1, _E), lambda b: (0, 0)),
        ],
        out_specs=[
            pl.BlockSpec((1, _N, _E), lambda b: (b, 0, 0)),
            pl.BlockSpec((1, 1, _E), lambda b: (b, 0, 0)),
            pl.BlockSpec((1, 1, _E), lambda b: (b, 0, 0)),
        ],
        out_shape=[
            jax.ShapeDtypeStruct((_B, _N, _E), jnp.float32),
            jax.ShapeDtypeStruct((_B, 1, _E), jnp.float32),
            jax.ShapeDtypeStruct((_B, 1, _E), jnp.float32),
        ],
    )(fe, stats, depot, W_dep, b_dep[None, :], bn_w[None, :], bn_b[None, :])

    h = jnp.concatenate([hd, hb], axis=1)
    return h, mh[:, 0, :]


# speedup vs baseline: 3.8363x; 1.0822x over previous
import jax
import jax.numpy as jnp
from jax.experimental import pallas as pl

_B, _N, _E = 16, 1000, 128
_RC = 1000
_NC = _N // _RC
_K = 10


def _knn_fe_body(xq_ref, xbT_ref, x0_ref, wc_ref, fe_ref, stats_ref):
    b = pl.program_id(0)
    c = pl.program_id(1)
    q0 = xq_ref[0, :, 0:1]
    q1 = xq_ref[0, :, 1:2]
    k0 = xbT_ref[0, 0:1, :]
    k1 = xbT_ref[0, 1:2, :]
    d0 = q0 - k0
    d1 = q1 - k1
    dist2 = d0 * d0 + d1 * d1
    iota = jax.lax.broadcasted_iota(jnp.int32, (_RC, _N), 1).astype(jnp.float32)
    big = jnp.float32(jnp.inf)
    bigi = jnp.float32(2e9)
    zero = jnp.float32(0.0)
    one = jnp.float32(1.0)

    def extract(work, m):
        cand = jnp.where(work == m, iota, bigi)
        idx = jnp.min(cand, axis=1, keepdims=True)
        return jnp.where(cand == idx, big, work)

    work = extract(dist2, zero)
    for _ in range(_K - 1):
        m = jnp.min(work, axis=1, keepdims=True)
        work = extract(work, m)

    acc = jnp.where(work == big, one, zero)
    s = jnp.dot(acc, x0_ref[0], preferred_element_type=jnp.float32)
    t0 = s[:, 0:1] - jnp.float32(_K) * q0
    t1 = s[:, 1:2] - jnp.float32(_K) * q1
    fe = (q0 * wc_ref[0:1, :] + q1 * wc_ref[1:2, :]
          + t0 * wc_ref[2:3, :] + t1 * wc_ref[3:4, :])
    fe_ref[0, :, :] = fe

    @pl.when((b == 0) & (c == 0))
    def _():
        stats_ref[:, :] = jnp.zeros((8, _E), jnp.float32)

    stats_ref[0:1, :] += jnp.sum(fe, axis=0, keepdims=True)
    stats_ref[1:2, :] += jnp.sum(fe * fe, axis=0, keepdims=True)


def _bn_body(fe_ref, stats_ref, dep_ref, wdep_ref, bdep_ref, bnw_ref, bnb_ref,
             hb_ref, hd_ref, mh_ref):
    inv_n = jnp.float32(1.0 / (_B * _N))
    mean = stats_ref[0:1, :] * inv_n
    ex2 = stats_ref[1:2, :] * inv_n
    var = ex2 - mean * mean
    scale = jax.lax.rsqrt(var + jnp.float32(1e-5)) * bnw_ref[0:1, :]
    fe = fe_ref[0]
    normed = (fe - mean) * scale + bnb_ref[0:1, :]
    hb = jnp.where(normed >= 0, normed, jnp.float32(0.01) * normed)
    hb_ref[0] = hb
    dd0 = dep_ref[0, :, 0:1]
    dd1 = dep_ref[0, :, 1:2]
    dep = dd0 * wdep_ref[0:1, :] + dd1 * wdep_ref[1:2, :] + bdep_ref[0:1, :]
    hd = jnp.where(dep >= 0, dep, jnp.float32(0.01) * dep)
    hd_ref[0] = hd
    mh_ref[0] = (jnp.sum(hb, axis=0, keepdims=True) + hd) / jnp.float32(_N + 1)


def kernel(loc, depot, W_init, b_init, W_nbr, b_nbr, W_fin, b_fin,
           W_dep, b_dep, bn_w, bn_b):
    locT = jnp.transpose(loc, (0, 2, 1))
    wc = jnp.concatenate([W_init @ W_fin, W_nbr @ W_fin], axis=0)

    fe, stats = pl.pallas_call(
        _knn_fe_body,
        grid=(_B, _NC),
        in_specs=[
            pl.BlockSpec((1, _RC, 2), lambda b, c: (b, c, 0)),
            pl.BlockSpec((1, 2, _N), lambda b, c: (b, 0, 0)),
            pl.BlockSpec((1, _N, 2), lambda b, c: (0, 0, 0)),
            pl.BlockSpec((4, _E), lambda b, c: (0, 0)),
        ],
        out_specs=[
            pl.BlockSpec((1, _RC, _E), lambda b, c: (b, c, 0)),
            pl.BlockSpec((8, _E), lambda b, c: (0, 0)),
        ],
        out_shape=[
            jax.ShapeDtypeStruct((_B, _N, _E), jnp.float32),
            jax.ShapeDtypeStruct((8, _E), jnp.float32),
        ],
    )(loc, locT, loc, wc)

    hb, hd, mh = pl.pallas_call(
        _bn_body,
        grid=(_B,),
        in_specs=[
            pl.BlockSpec((1, _N, _E), lambda b: (b, 0, 0)),
            pl.BlockSpec((8, _E), lambda b: (0, 0)),
            pl.BlockSpec((1, 1, 2), lambda b: (b, 0, 0)),
            pl.BlockSpec((2, _E), lambda b: (0, 0)),
            pl.BlockSpec((1, _E), lambda b: (0, 0)),
            pl.BlockSpec((1, _E), lambda b: (0, 0)),
            pl.BlockSpec((1, _E), lambda b: (0, 0)),
        ],
        out_specs=[
            pl.BlockSpec((1, _N, _E), lambda b: (b, 0, 0)),
            pl.BlockSpec((1, 1, _E), lambda b: (b, 0, 0)),
            pl.BlockSpec((1, 1, _E), lambda b: (b, 0, 0)),
        ],
        out_shape=[
            jax.ShapeDtypeStruct((_B, _N, _E), jnp.float32),
            jax.ShapeDtypeStruct((_B, 1, _E), jnp.float32),
            jax.ShapeDtypeStruct((_B, 1, _E), jnp.float32),
        ],
    )(fe, stats, depot, W_dep, b_dep[None, :], bn_w[None, :], bn_b[None, :])

    h = jnp.concatenate([hd, hb], axis=1)
    return h, mh[:, 0, :]
